# baseline (speedup 1.0000x reference)
import jax
import jax.numpy as jnp
from jax import lax
from jax.experimental import pallas as pl
from jax.experimental.pallas import tpu as pltpu

_K = 3
_CBC = 128


def _nms_chunk(x, lane, row):
    h, w = x.shape[-2], x.shape[-1]
    xl = jnp.where(lane == w - 1, 0.0, pltpu.roll(x, w - 1, axis=2))
    xr = jnp.where(lane == 0, 0.0, pltpu.roll(x, 1, axis=2))
    m = jnp.maximum(jnp.maximum(x, xl), xr)
    mu = jnp.where(row == h - 1, 0.0, pltpu.roll(m, h - 1, axis=1))
    md = jnp.where(row == 0, 0.0, pltpu.roll(m, 1, axis=1))
    lm = jnp.maximum(jnp.maximum(m, mu), md)
    return jnp.where(x == lm, x, 0.0)


def _nms_kernel(x_hbm, o_hbm, x_buf, o_buf, in_sem, out_sem):
    n, h, w = x_hbm.shape
    n_steps = n // _CBC
    lane = lax.broadcasted_iota(jnp.int32, (1, 1, w), 2)
    row = lax.broadcasted_iota(jnp.int32, (1, h, 1), 1)

    def dma_in(slot, step):
        pltpu.make_async_copy(
            x_hbm.at[pl.ds(step * _CBC, _CBC)], x_buf.at[slot], in_sem.at[slot]
        ).start()

    def wait_in(slot):
        pltpu.make_async_copy(
            x_hbm.at[pl.ds(0, _CBC)], x_buf.at[slot], in_sem.at[slot]
        ).wait()

    def dma_out(slot, step):
        pltpu.make_async_copy(
            o_buf.at[slot], o_hbm.at[pl.ds(step * _CBC, _CBC)], out_sem.at[slot]
        ).start()

    def wait_out(slot):
        pltpu.make_async_copy(
            o_buf.at[slot], o_hbm.at[pl.ds(0, _CBC)], out_sem.at[slot]
        ).wait()

    for s in range(_K - 1):
        dma_in(s, s)

    for step in range(n_steps):
        cur = step % _K
        if step + _K - 1 < n_steps:
            dma_in((step + _K - 1) % _K, step + _K - 1)
        wait_in(cur)
        if step >= _K:
            wait_out(cur)
        for i0 in range(0, _CBC, 1):
            x = x_buf[cur, i0 : i0 + 1]
            o_buf[cur, i0 : i0 + 1] = _nms_chunk(x, lane, row)
        dma_out(cur, step)

    for s in range(_K):
        wait_out((n_steps - _K + s) % _K)


def kernel(points):
    b, c, h, w = points.shape
    flat = points.reshape(b * c, h, w)
    out = pl.pallas_call(
        _nms_kernel,
        out_shape=jax.ShapeDtypeStruct(flat.shape, flat.dtype),
        in_specs=[pl.BlockSpec(memory_space=pl.ANY)],
        out_specs=pl.BlockSpec(memory_space=pl.ANY),
        scratch_shapes=[
            pltpu.VMEM((_K, _CBC, h, w), jnp.float32),
            pltpu.VMEM((_K, _CBC, h, w), jnp.float32),
            pltpu.SemaphoreType.DMA((_K,)),
            pltpu.SemaphoreType.DMA((_K,)),
        ],
    )(flat)
    return out.reshape(b, c, h, w)


# speedup vs baseline: 1.0621x; 1.0621x over previous
import jax
import jax.numpy as jnp
from jax import lax
from jax.experimental import pallas as pl
from jax.experimental.pallas import tpu as pltpu

_K = 3
_CBC = 128
_P = 4


def _nms_chunk(x, lane, row):
    h, w = x.shape[-2], x.shape[-1]
    xl = jnp.where(lane == w - 1, 0.0, pltpu.roll(x, w - 1, axis=2))
    xr = jnp.where(lane == 0, 0.0, pltpu.roll(x, 1, axis=2))
    m = jnp.maximum(jnp.maximum(x, xl), xr)
    mu = jnp.where(row == h - 1, 0.0, pltpu.roll(m, h - 1, axis=1))
    md = jnp.where(row == 0, 0.0, pltpu.roll(m, 1, axis=1))
    lm = jnp.maximum(jnp.maximum(m, mu), md)
    return jnp.where(x == lm, x, 0.0)


def _nms_kernel(x_hbm, o_hbm, x_buf, o_buf, in_sem, out_sem):
    n, h, w = x_hbm.shape
    n_steps = n // _CBC
    lane = lax.broadcasted_iota(jnp.int32, (1, 1, w), 2)
    row = lax.broadcasted_iota(jnp.int32, (1, h, 1), 1)

    pc = _CBC // _P

    def dma_in(slot, step, part):
        pltpu.make_async_copy(
            x_hbm.at[pl.ds(step * _CBC + part * pc, pc)],
            x_buf.at[slot, pl.ds(part * pc, pc)],
            in_sem.at[slot, part],
        ).start()

    def wait_in(slot, part):
        pltpu.make_async_copy(
            x_hbm.at[pl.ds(0, pc)],
            x_buf.at[slot, pl.ds(0, pc)],
            in_sem.at[slot, part],
        ).wait()

    def dma_out(slot, step, part):
        pltpu.make_async_copy(
            o_buf.at[slot, pl.ds(part * pc, pc)],
            o_hbm.at[pl.ds(step * _CBC + part * pc, pc)],
            out_sem.at[slot, part],
        ).start()

    def wait_out(slot, part):
        pltpu.make_async_copy(
            o_buf.at[slot, pl.ds(0, pc)],
            o_hbm.at[pl.ds(0, pc)],
            out_sem.at[slot, part],
        ).wait()

    for s in range(_K - 1):
        for p in range(_P):
            dma_in(s, s, p)

    def body(step, _):
        cur = lax.rem(step, _K)
        @pl.when(step + _K - 1 < n_steps)
        def _():
            for p in range(_P):
                dma_in(lax.rem(step + _K - 1, _K), step + _K - 1, p)
        @pl.when(step >= _K)
        def _():
            for p in range(_P):
                wait_out(cur, p)
        for p in range(_P):
            wait_in(cur, p)
            for i0 in range(p * pc, (p + 1) * pc):
                x = x_buf[cur, i0 : i0 + 1]
                o_buf[cur, i0 : i0 + 1] = _nms_chunk(x, lane, row)
            dma_out(cur, step, p)
        return ()

    lax.fori_loop(0, n_steps, body, ())
    for s in range(_K):
        for p in range(_P):
            wait_out(lax.rem(n_steps - _K + s, _K), p)


def kernel(points):
    b, c, h, w = points.shape
    flat = points.reshape(b * c, h, w)
    out = pl.pallas_call(
        _nms_kernel,
        out_shape=jax.ShapeDtypeStruct(flat.shape, flat.dtype),
        in_specs=[pl.BlockSpec(memory_space=pl.ANY)],
        out_specs=pl.BlockSpec(memory_space=pl.ANY),
        scratch_shapes=[
            pltpu.VMEM((_K, _CBC, h, w), jnp.float32),
            pltpu.VMEM((_K, _CBC, h, w), jnp.float32),
            pltpu.SemaphoreType.DMA((_K, _P)),
            pltpu.SemaphoreType.DMA((_K, _P)),
        ],
    )(flat)
    return out.reshape(b, c, h, w)
